# Initial kernel scaffold; baseline (speedup 1.0000x reference)
import jax
import jax.numpy as jnp
from jax.experimental import pallas as pl


def kernel(x, codebooks):
    raise NotImplementedError("write your pallas kernel here")


# speedup vs baseline: 1.9522x; 1.9522x over previous
import jax
import jax.numpy as jnp
from jax.experimental import pallas as pl
from jax.experimental.pallas import tpu as pltpu

_TN = 1024
_EPS_LOG = 1e-10


def _vq_body(x_ref, cb_ref, out_ref, idx_ref, perp_ref, acc_ref):
    m, d = cb_ref.shape[1], cb_ref.shape[2]
    tn = x_ref.shape[2]
    r_stages = idx_ref.shape[0]
    b = pl.program_id(1)
    t = pl.program_id(2)
    nb = pl.num_programs(1)
    nt = pl.num_programs(2)

    @pl.when(jnp.logical_and(b == 0, t == 0))
    def _init():
        acc_ref[...] = jnp.zeros_like(acc_ref)

    cb = cb_ref[0]
    norm = jnp.sqrt(jnp.sum(cb * cb, axis=1, keepdims=True))
    c2 = (cb / jnp.maximum(norm, 1e-12)).astype(jnp.bfloat16)
    cb_lo = cb.astype(jnp.bfloat16)

    xv = x_ref[0]
    iota_m = jax.lax.broadcasted_iota(jnp.int32, (m, tn), 0)

    resid = xv
    total = jnp.zeros_like(xv)
    for r in range(r_stages):
        qn = jnp.sqrt(jnp.sum(resid * resid, axis=0, keepdims=True))
        q2 = (resid / jnp.maximum(qn, 1e-12)).astype(jnp.bfloat16)
        sim = jax.lax.dot_general(
            c2, q2, (((1,), (0,)), ((), ())),
            preferred_element_type=jnp.float32)
        mx = jnp.max(sim, axis=0, keepdims=True)
        idx = jnp.min(jnp.where(sim == mx, iota_m, m), axis=0)
        onehot = (iota_m == idx[None, :]).astype(jnp.bfloat16)
        quant = jax.lax.dot_general(
            cb_lo, onehot, (((0,), (0,)), ((), ())),
            preferred_element_type=jnp.float32)
        resid = resid - quant
        total = total + quant
        idx_ref[r, 0, 0, 0, :] = idx
        acc_ref[:, r:r + 1] += jnp.sum(
            onehot.astype(jnp.float32), axis=1, keepdims=True)
    out_ref[0] = total

    @pl.when(jnp.logical_and(b == nb - 1, t == nt - 1))
    def _finalize():
        mean = acc_ref[...] / (nb * nt * tn)
        ent = -jnp.sum(mean * jnp.log(mean + _EPS_LOG), axis=0, keepdims=True)
        perp_ref[0] = jnp.exp(ent)


def kernel(x, codebooks):
    bsz, chan, tlen = x.shape
    h, m, d = codebooks.shape
    r_stages = 2
    nt = tlen // _TN
    out, idx, perp = pl.pallas_call(
        _vq_body,
        grid=(h, bsz, nt),
        in_specs=[
            pl.BlockSpec((1, d, _TN), lambda hh, bb, tt: (bb, hh, tt)),
            pl.BlockSpec((1, m, d), lambda hh, bb, tt: (hh, 0, 0)),
        ],
        out_specs=[
            pl.BlockSpec((1, d, _TN), lambda hh, bb, tt: (bb, hh, tt)),
            pl.BlockSpec((r_stages, 1, 1, 1, _TN),
                         lambda hh, bb, tt: (0, bb, hh, 0, tt)),
            pl.BlockSpec((1, 1, r_stages), lambda hh, bb, tt: (hh, 0, 0)),
        ],
        out_shape=[
            jax.ShapeDtypeStruct((bsz, chan, tlen), jnp.float32),
            jax.ShapeDtypeStruct((r_stages, bsz, h, 1, tlen), jnp.int32),
            jax.ShapeDtypeStruct((h, 1, r_stages), jnp.float32),
        ],
        scratch_shapes=[pltpu.VMEM((m, r_stages), jnp.float32)],
    )(x, codebooks)
    indices = jnp.transpose(idx.reshape(r_stages, bsz, h, tlen), (1, 2, 3, 0))
    perplexity = perp.reshape(h * r_stages)
    return out, indices, perplexity
